# Initial kernel scaffold; baseline (speedup 1.0000x reference)
import jax
import jax.numpy as jnp
from jax.experimental import pallas as pl


def kernel(x, Wg, WA, WB):
    raise NotImplementedError("write your pallas kernel here")


# speedup vs baseline: 8.2955x; 8.2955x over previous
import functools

import jax
import jax.numpy as jnp
from jax.experimental import pallas as pl
from jax.experimental.pallas import tpu as pltpu

E = 8
R = 64
D = 2048
SCALING = 4.0

BM = 512


def _fused_dense_kernel(x_ref, wg_ref, wa_ref, wb_ref, o_ref):
    x = x_ref[...]
    logits = jax.lax.dot_general(x, wg_ref[...], (((1,), (1,)), ((), ())),
                                 preferred_element_type=jnp.float32)
    maxv = jnp.max(logits, axis=1, keepdims=True)
    denom = jnp.sum(jnp.exp(logits - maxv), axis=1, keepdims=True)
    p = 1.0 / denom
    eidx = jax.lax.broadcasted_iota(jnp.int32, logits.shape, 1)
    gate = jnp.min(jnp.where(logits >= maxv, eidx, E), axis=1, keepdims=True)

    h = jax.lax.dot_general(x, wa_ref[...], (((1,), (1,)), ((), ())),
                            preferred_element_type=jnp.float32)
    col_e = jax.lax.broadcasted_iota(jnp.int32, h.shape, 1) // R
    hm = jnp.where(col_e == gate, h * (SCALING * p), 0.0)
    o_ref[...] = jax.lax.dot_general(hm, wb_ref[...], (((1,), (0,)), ((), ())),
                                     preferred_element_type=jnp.float32)


@jax.jit
def kernel(x, Wg, WA, WB):
    bsz, seq, d = x.shape
    T = bsz * seq
    xf = x.reshape(T, d)
    WA_all = WA.reshape(E * R, D)
    WB_stack = WB.transpose(0, 2, 1).reshape(E * R, D)

    out = pl.pallas_call(
        _fused_dense_kernel,
        grid=(T // BM,),
        in_specs=[
            pl.BlockSpec((BM, D), lambda i: (i, 0)),
            pl.BlockSpec((E, D), lambda i: (0, 0)),
            pl.BlockSpec((E * R, D), lambda i: (0, 0)),
            pl.BlockSpec((E * R, D), lambda i: (0, 0)),
        ],
        out_specs=pl.BlockSpec((BM, D), lambda i: (i, 0)),
        out_shape=jax.ShapeDtypeStruct((T, D), jnp.float32),
    )(xf, Wg, WA_all, WB_stack)
    return out.reshape(bsz, seq, d)
